# Initial kernel scaffold; baseline (speedup 1.0000x reference)
import jax
import jax.numpy as jnp
from jax.experimental import pallas as pl


def kernel(x, edge_index):
    raise NotImplementedError("write your pallas kernel here")


# speedup vs baseline: 3.2664x; 3.2664x over previous
import functools

import jax
import jax.numpy as jnp
from jax import lax
from jax.experimental import pallas as pl
from jax.experimental.pallas import tpu as pltpu
from jax.experimental.pallas import tpu_sc as plsc

N = 10000
E = 320000
D = 128
W = 128
NC = 2
NS = 16
NW = NC * NS
CHUNKS_PER_TILE = 80
N_CHUNKS = NW * CHUNKS_PER_TILE
E_PAD = N_CHUNKS * W
ZROWS = 632
ACC_ROWS = NS * ZROWS
OROWS = 624
OROWS_LAST = N - 15 * OROWS


def _sc_gather_scatter_add(x, src2d, dst2d, zeros_init):
    mesh = plsc.VectorSubcoreMesh(core_axis_name="c", subcore_axis_name="s")

    @functools.partial(
        pl.kernel,
        out_type=jax.ShapeDtypeStruct((NC, N, D), jnp.float32),
        mesh=mesh,
        scratch_types=[
            pltpu.VMEM((CHUNKS_PER_TILE, W), jnp.int32),
            pltpu.VMEM((CHUNKS_PER_TILE, W), jnp.int32),
            pltpu.VMEM((W, D), jnp.float32),
            pltpu.VMEM_SHARED((ACC_ROWS, D), jnp.float32),
            pltpu.SemaphoreType.DMA,
        ],
    )
    def k(x_hbm, src_hbm, dst_hbm, z_hbm, out_hbm, src_v, dst_v, rows_v, acc,
          sem):
        c = lax.axis_index("c")
        s = lax.axis_index("s")
        w = c * NS + s

        pltpu.sync_copy(z_hbm, acc.at[pl.ds(s * ZROWS, ZROWS)])

        lo = w * CHUNKS_PER_TILE
        pltpu.sync_copy(src_hbm.at[pl.ds(lo, CHUNKS_PER_TILE)], src_v)
        pltpu.sync_copy(dst_hbm.at[pl.ds(lo, CHUNKS_PER_TILE)], dst_v)

        plsc.subcore_barrier()

        @pl.loop(0, CHUNKS_PER_TILE)
        def _(j):
            pltpu.async_copy(x_hbm.at[src_v.at[j]], rows_v, sem).wait()
            pltpu.sync_copy(rows_v, acc.at[dst_v.at[j]], add=True)

        plsc.subcore_barrier()

        ob = s * OROWS

        @pl.when(s < NS - 1)
        def _():
            pltpu.sync_copy(acc.at[pl.ds(ob, OROWS)],
                            out_hbm.at[c].at[pl.ds(ob, OROWS)])

        @pl.when(s == NS - 1)
        def _():
            pltpu.sync_copy(acc.at[pl.ds((NS - 1) * OROWS, OROWS_LAST)],
                            out_hbm.at[c].at[pl.ds((NS - 1) * OROWS, OROWS_LAST)])

    return k(x, src2d, dst2d, zeros_init)


def _combine_partials(partials):
    blk = 1000

    def body(p_ref, o_ref):
        o_ref[...] = p_ref[0] + p_ref[1]

    return pl.pallas_call(
        body,
        out_shape=jax.ShapeDtypeStruct((N, D), jnp.float32),
        grid=(N // blk,),
        in_specs=[pl.BlockSpec((2, blk, D), lambda i: (0, i, 0))],
        out_specs=pl.BlockSpec((blk, D), lambda i: (i, 0)),
    )(partials)


def kernel(x, edge_index):
    src = edge_index[0].astype(jnp.int32)
    dst = edge_index[1].astype(jnp.int32)
    pad = E_PAD - E
    src2d = jnp.concatenate([src, jnp.zeros((pad,), jnp.int32)]).reshape(N_CHUNKS, W)
    dst2d = jnp.concatenate([dst, jnp.full((pad,), N, jnp.int32)]).reshape(N_CHUNKS, W)
    zeros_init = jnp.zeros((ZROWS, D), jnp.float32)
    partials = _sc_gather_scatter_add(x, src2d, dst2d, zeros_init)
    return _combine_partials(partials)


# speedup vs baseline: 3.5818x; 1.0966x over previous
import functools

import jax
import jax.numpy as jnp
from jax import lax
from jax.experimental import pallas as pl
from jax.experimental.pallas import tpu as pltpu
from jax.experimental.pallas import tpu_sc as plsc

N = 10000
E = 320000
D = 128
W = 128
NC = 2
NS = 16
NW = NC * NS
CHUNKS_PER_TILE = 80
N_CHUNKS = NW * CHUNKS_PER_TILE
E_PAD = N_CHUNKS * W
ZROWS = 632
ACC_ROWS = NS * ZROWS
OROWS = 624
OROWS_LAST = N - 15 * OROWS
HALF = CHUNKS_PER_TILE // 2


def _sc_gather_scatter_add(x, src2d, dst2d, zeros_init):
    mesh = plsc.VectorSubcoreMesh(core_axis_name="c", subcore_axis_name="s")

    @functools.partial(
        pl.kernel,
        out_type=jax.ShapeDtypeStruct((NC, N, D), jnp.float32),
        mesh=mesh,
        scratch_types=[
            pltpu.VMEM((HALF, W), jnp.int32),
            pltpu.VMEM((HALF, W), jnp.int32),
            pltpu.VMEM((2, W, D), jnp.float32),
            pltpu.VMEM_SHARED((ACC_ROWS, D), jnp.float32),
            pltpu.SemaphoreType.DMA,
            pltpu.SemaphoreType.DMA,
        ],
    )
    def k(x_hbm, src_hbm, dst_hbm, z_hbm, out_hbm, src_v, dst_v, rows_v, acc,
          sem0, sem1):
        c = lax.axis_index("c")
        s = lax.axis_index("s")
        w = c * NS + s

        zcopy = pltpu.async_copy(z_hbm, acc.at[pl.ds(s * ZROWS, ZROWS)], sem1)

        lo = w * CHUNKS_PER_TILE
        zcopy.wait()
        plsc.subcore_barrier()

        for p in range(CHUNKS_PER_TILE // HALF):
            pltpu.sync_copy(src_hbm.at[pl.ds(lo + p * HALF, HALF)], src_v)
            pltpu.sync_copy(dst_hbm.at[pl.ds(lo + p * HALF, HALF)], dst_v)

            pltpu.async_copy(x_hbm.at[src_v.at[0]], rows_v.at[0], sem0)
            pltpu.async_copy(x_hbm.at[src_v.at[1]], rows_v.at[1], sem1)

            @pl.loop(0, HALF, step=2)
            def _(j):
                pltpu.make_async_copy(x_hbm.at[src_v.at[j]], rows_v.at[0],
                                      sem0).wait()
                pltpu.sync_copy(rows_v.at[0], acc.at[dst_v.at[j]], add=True)

                @pl.when(j + 2 < HALF)
                def _():
                    pltpu.async_copy(x_hbm.at[src_v.at[j + 2]], rows_v.at[0],
                                     sem0)

                pltpu.make_async_copy(x_hbm.at[src_v.at[j + 1]], rows_v.at[1],
                                      sem1).wait()
                pltpu.sync_copy(rows_v.at[1], acc.at[dst_v.at[j + 1]], add=True)

                @pl.when(j + 3 < HALF)
                def _():
                    pltpu.async_copy(x_hbm.at[src_v.at[j + 3]], rows_v.at[1],
                                     sem1)

        plsc.subcore_barrier()

        ob = s * OROWS

        @pl.when(s < NS - 1)
        def _():
            pltpu.sync_copy(acc.at[pl.ds(ob, OROWS)],
                            out_hbm.at[c].at[pl.ds(ob, OROWS)])

        @pl.when(s == NS - 1)
        def _():
            pltpu.sync_copy(acc.at[pl.ds((NS - 1) * OROWS, OROWS_LAST)],
                            out_hbm.at[c].at[pl.ds((NS - 1) * OROWS, OROWS_LAST)])

    return k(x, src2d, dst2d, zeros_init)


def _combine_partials(partials):
    blk = 1000

    def body(p_ref, o_ref):
        o_ref[...] = p_ref[0] + p_ref[1]

    return pl.pallas_call(
        body,
        out_shape=jax.ShapeDtypeStruct((N, D), jnp.float32),
        grid=(N // blk,),
        in_specs=[pl.BlockSpec((2, blk, D), lambda i: (0, i, 0))],
        out_specs=pl.BlockSpec((blk, D), lambda i: (i, 0)),
    )(partials)


def kernel(x, edge_index):
    src = edge_index[0].astype(jnp.int32)
    dst = edge_index[1].astype(jnp.int32)
    pad = E_PAD - E
    src2d = jnp.concatenate([src, jnp.zeros((pad,), jnp.int32)]).reshape(N_CHUNKS, W)
    dst2d = jnp.concatenate([dst, jnp.full((pad,), N, jnp.int32)]).reshape(N_CHUNKS, W)
    zeros_init = jnp.zeros((ZROWS, D), jnp.float32)
    partials = _sc_gather_scatter_add(x, src2d, dst2d, zeros_init)
    return _combine_partials(partials)


# speedup vs baseline: 12.6593x; 3.5343x over previous
import functools

import jax
import jax.numpy as jnp
from jax import lax
from jax.experimental import pallas as pl
from jax.experimental.pallas import tpu as pltpu
from jax.experimental.pallas import tpu_sc as plsc

N = 10000
E = 320000
D = 128
W = 128
NC = 2
NS = 16
NW = NC * NS
CHUNKS_PER_TILE = 80
N_CHUNKS = NW * CHUNKS_PER_TILE
E_PAD = N_CHUNKS * W
ZROWS = 632
ACC_ROWS = NS * ZROWS
OROWS = 624
OROWS_LAST = N - 15 * OROWS
HALF = CHUNKS_PER_TILE // 2


def _sc_gather_scatter_add(x, src2d, dst2d, zeros_init):
    mesh = plsc.VectorSubcoreMesh(core_axis_name="c", subcore_axis_name="s")

    @functools.partial(
        pl.kernel,
        out_type=jax.ShapeDtypeStruct((NC, N, D), jnp.float32),
        mesh=mesh,
        scratch_types=[
            pltpu.VMEM((HALF, W), jnp.int32),
            pltpu.VMEM((HALF, W), jnp.int32),
            pltpu.VMEM((2, W, D), jnp.float32),
            pltpu.VMEM_SHARED((ACC_ROWS, D), jnp.float32),
            pltpu.SemaphoreType.DMA,
            pltpu.SemaphoreType.DMA,
        ],
    )
    def k(x_hbm, src_hbm, dst_hbm, z_hbm, out_hbm, src_v, dst_v, rows_v, acc,
          sem0, sem1):
        c = lax.axis_index("c")
        s = lax.axis_index("s")
        w = c * NS + s

        zcopy = pltpu.async_copy(z_hbm, acc.at[pl.ds(s * ZROWS, ZROWS)], sem1)

        lo = w * CHUNKS_PER_TILE
        zcopy.wait()
        plsc.subcore_barrier()

        for p in range(CHUNKS_PER_TILE // HALF):
            pltpu.sync_copy(src_hbm.at[pl.ds(lo + p * HALF, HALF)], src_v)
            pltpu.sync_copy(dst_hbm.at[pl.ds(lo + p * HALF, HALF)], dst_v)

            pltpu.async_copy(x_hbm.at[src_v.at[0]], rows_v.at[0], sem0)
            pltpu.async_copy(x_hbm.at[src_v.at[1]], rows_v.at[1], sem1)

            @pl.loop(0, HALF, step=2)
            def _(j):
                pltpu.make_async_copy(x_hbm.at[src_v.at[j]], rows_v.at[0],
                                      sem0).wait()
                pltpu.sync_copy(rows_v.at[0], acc.at[dst_v.at[j]], add=True)

                @pl.when(j + 2 < HALF)
                def _():
                    pltpu.async_copy(x_hbm.at[src_v.at[j + 2]], rows_v.at[0],
                                     sem0)

                pltpu.make_async_copy(x_hbm.at[src_v.at[j + 1]], rows_v.at[1],
                                      sem1).wait()
                pltpu.sync_copy(rows_v.at[1], acc.at[dst_v.at[j + 1]], add=True)

                @pl.when(j + 3 < HALF)
                def _():
                    pltpu.async_copy(x_hbm.at[src_v.at[j + 3]], rows_v.at[1],
                                     sem1)

        plsc.subcore_barrier()

        ob = s * OROWS

        @pl.when(s < NS - 1)
        def _():
            pltpu.sync_copy(acc.at[pl.ds(ob, OROWS)],
                            out_hbm.at[c].at[pl.ds(ob, OROWS)])

        @pl.when(s == NS - 1)
        def _():
            pltpu.sync_copy(acc.at[pl.ds((NS - 1) * OROWS, OROWS_LAST)],
                            out_hbm.at[c].at[pl.ds((NS - 1) * OROWS, OROWS_LAST)])

    return k(x, src2d, dst2d, zeros_init)


def _combine_partials(partials):
    blk = 1000

    def body(p_ref, o_ref):
        o_ref[...] = p_ref[0] + p_ref[1]

    return pl.pallas_call(
        body,
        out_shape=jax.ShapeDtypeStruct((N, D), jnp.float32),
        grid=(N // blk,),
        in_specs=[pl.BlockSpec((2, blk, D), lambda i: (0, i, 0))],
        out_specs=pl.BlockSpec((blk, D), lambda i: (i, 0)),
    )(partials)


def kernel(x, edge_index):
    src = edge_index[0].astype(jnp.int32)
    dst = edge_index[1].astype(jnp.int32)
    pad = E_PAD - E
    pad_ids = jnp.arange(pad, dtype=jnp.int32)
    src2d = jnp.concatenate([src, pad_ids % N]).reshape(N_CHUNKS, W)
    dst2d = jnp.concatenate([dst, N + pad_ids % (ACC_ROWS - N)]).reshape(N_CHUNKS, W)
    zeros_init = jnp.zeros((ZROWS, D), jnp.float32)
    partials = _sc_gather_scatter_add(x, src2d, dst2d, zeros_init)
    return _combine_partials(partials)
